# Initial kernel scaffold; baseline (speedup 1.0000x reference)
import functools

import jax
import jax.numpy as jnp
from jax import lax
from jax.experimental import pallas as pl
from jax.experimental.pallas import tpu as pltpu
from jax.experimental.pallas import tpu_sc as plsc

N = 10000
NPAD = 10240
D = 128
NCLS = 80
NC, NS = 2, 16
NW = NC * NS
CHUNK = 128
ROWS_PER_TILE = NPAD // NS

_mesh = plsc.VectorSubcoreMesh(core_axis_name="c", subcore_axis_name="s")


def _deg_body(chunks, dst_hbm, w_hbm, deg_hbm, dst_v, w_v, deg_sh, zbuf):
    c = lax.axis_index("c")
    s = lax.axis_index("s")
    wid = c * NS + s
    pltpu.sync_copy(dst_hbm.at[wid], dst_v)
    pltpu.sync_copy(w_hbm.at[wid], w_v)

    def zrow(i, carry):
        zbuf[pl.ds(i * 16, 16)] = jnp.zeros((16,), jnp.float32)
        return carry

    lax.fori_loop(0, ROWS_PER_TILE // 16, zrow, 0)
    pltpu.sync_copy(zbuf, deg_sh.at[pl.ds(s * ROWS_PER_TILE, ROWS_PER_TILE)])
    plsc.subcore_barrier()

    def chunk(j, carry):
        pltpu.sync_copy(w_v.at[j], deg_sh.at[dst_v.at[j]], add=True)
        return carry

    lax.fori_loop(0, chunks, chunk, 0)
    plsc.subcore_barrier()
    pltpu.sync_copy(deg_sh.at[pl.ds(s * ROWS_PER_TILE, ROWS_PER_TILE)],
                    deg_hbm.at[c, pl.ds(s * ROWS_PER_TILE, ROWS_PER_TILE)])


def _make_deg(chunks):
    return pl.kernel(
        functools.partial(_deg_body, chunks),
        out_type=jax.ShapeDtypeStruct((NC, NPAD), jnp.float32),
        mesh=_mesh,
        scratch_types=[
            pltpu.VMEM((chunks, CHUNK), jnp.int32),
            pltpu.VMEM((chunks, CHUNK), jnp.float32),
            pltpu.VMEM_SHARED((NPAD,), jnp.float32),
            pltpu.VMEM((ROWS_PER_TILE,), jnp.float32),
        ],
    )


def _edge_body(chunks, y_hbm, src_hbm, dst_hbm, w_hbm, z_hbm,
               src_v, dst_v, w_v, rbuf, acc, gsem):
    c = lax.axis_index("c")
    s = lax.axis_index("s")
    wid = c * NS + s
    pltpu.sync_copy(src_hbm.at[wid], src_v)
    pltpu.sync_copy(dst_hbm.at[wid], dst_v)
    pltpu.sync_copy(w_hbm.at[wid], w_v)

    def zrow(i, carry):
        for t in range(8):
            rbuf[i, pl.ds(t * 16, 16)] = jnp.zeros((16,), jnp.float32)
        return carry

    lax.fori_loop(0, CHUNK, zrow, 0)
    for k in range(ROWS_PER_TILE // CHUNK):
        pltpu.sync_copy(
            rbuf, acc.at[pl.ds(s * ROWS_PER_TILE + k * CHUNK, CHUNK)])
    plsc.subcore_barrier()

    def chunk(j, carry):
        pltpu.async_copy(y_hbm.at[src_v.at[j]], rbuf, gsem).wait()

        def scale(k, c2):
            wk = w_v[j, k]
            for t in range(8):
                sl = pl.ds(t * 16, 16)
                rbuf[k, sl] = rbuf[k, sl] * wk
            return c2

        lax.fori_loop(0, CHUNK, scale, 0)
        pltpu.sync_copy(rbuf, acc.at[dst_v.at[j]], add=True)
        return carry

    lax.fori_loop(0, chunks, chunk, 0)
    plsc.subcore_barrier()
    pltpu.sync_copy(acc.at[pl.ds(s * ROWS_PER_TILE, ROWS_PER_TILE)],
                    z_hbm.at[c, pl.ds(s * ROWS_PER_TILE, ROWS_PER_TILE)])


def _make_edge(chunks):
    return pl.kernel(
        functools.partial(_edge_body, chunks),
        out_type=jax.ShapeDtypeStruct((NC, NPAD, D), jnp.float32),
        mesh=_mesh,
        scratch_types=[
            pltpu.VMEM((chunks, CHUNK), jnp.int32),
            pltpu.VMEM((chunks, CHUNK), jnp.int32),
            pltpu.VMEM((chunks, CHUNK), jnp.float32),
            pltpu.VMEM((CHUNK, D), jnp.float32),
            pltpu.VMEM_SHARED((NPAD, D), jnp.float32),
            pltpu.SemaphoreType.DMA,
        ],
    )


def _dinv_from(degp):
    deg = degp[0] + degp[1] + 1.0
    return jnp.where(deg > 0, lax.rsqrt(deg), 0.0)[:N]


def _front_body(boxes_r, scores_r, labels_r, invwh_r, emb_r, w1g_r, w1e_r,
                b1_r, w2_r, b2_r, degp_r, g0w_r, x_out, y0_out):
    bx = boxes_r[...]
    x1, y1, x2, y2 = bx[:, 0:1], bx[:, 1:2], bx[:, 2:3], bx[:, 3:4]
    w = jnp.maximum(x2 - x1, 1.0)
    h = jnp.maximum(y2 - y1, 1.0)
    inv = invwh_r[...]
    invW, invH = inv[0:1, 0:1], inv[0:1, 1:2]
    cxn = (x1 + x2) * 0.5 * invW
    cyn = (y1 + y2) * 0.5 * invH
    wn = w * invW
    hn = h * invH
    cols = (cxn, cyn, wn, hn, wn * hn, w / (h + 1e-6), scores_r[...])
    g = w1g_r[...]
    pre = cols[0] * g[0:1]
    for t in range(1, 7):
        pre = pre + cols[t] * g[t:t + 1]
    lab = labels_r[...]
    io = lax.broadcasted_iota(jnp.int32, (N, NCLS), 1)
    oh = jnp.where(io == lab, 1.0, 0.0)
    embw = jnp.dot(emb_r[...], w1e_r[...], preferred_element_type=jnp.float32)
    pre = pre + jnp.dot(oh, embw, preferred_element_type=jnp.float32) + b1_r[...]
    xx = jnp.maximum(pre, 0.0)
    xx = jnp.maximum(
        jnp.dot(xx, w2_r[...], preferred_element_type=jnp.float32) + b2_r[...],
        0.0)
    dinv = _dinv_from(degp_r[...])
    x_out[...] = xx
    y0_out[...] = jnp.dot(xx, g0w_r[...],
                          preferred_element_type=jnp.float32) * dinv


_front = pl.pallas_call(
    _front_body,
    out_shape=(jax.ShapeDtypeStruct((N, D), jnp.float32),
               jax.ShapeDtypeStruct((N, D), jnp.float32)),
)


def _mid_body(zp_r, yp_r, x_r, degp_r, b_r, wn_r, xn_out, yn_out):
    dinv = _dinv_from(degp_r[...])
    zp = zp_r[...]
    z = zp[0, :N] + zp[1, :N]
    conv = dinv * (z + yp_r[...]) + b_r[...]
    xn = jnp.maximum(conv, 0.0) + x_r[...]
    xn_out[...] = xn
    yn_out[...] = jnp.dot(xn, wn_r[...],
                          preferred_element_type=jnp.float32) * dinv


_mid = pl.pallas_call(
    _mid_body,
    out_shape=(jax.ShapeDtypeStruct((N, D), jnp.float32),
               jax.ShapeDtypeStruct((N, D), jnp.float32)),
)


def _tail_body(zp_r, yp_r, x_r, degp_r, b_r, hw_r, hb_r, out):
    dinv = _dinv_from(degp_r[...])
    zp = zp_r[...]
    z = zp[0, :N] + zp[1, :N]
    conv = dinv * (z + yp_r[...]) + b_r[...]
    xn = jnp.maximum(conv, 0.0) + x_r[...]
    out[...] = jnp.dot(xn, hw_r[...],
                       preferred_element_type=jnp.float32) + hb_r[...]


_tail = pl.pallas_call(
    _tail_body,
    out_shape=jax.ShapeDtypeStruct((N, 4), jnp.float32),
)


def kernel(boxes, scores, labels, H, W, edge_index, edge_weight,
           emb, mlp_w1, mlp_b1, mlp_w2, mlp_b2,
           gcn_w0, gcn_b0, gcn_w1, gcn_b1, gcn_w2, gcn_b2,
           head_w, head_b):
    f32 = jnp.float32
    src = edge_index[0].astype(jnp.int32)
    dst = edge_index[1].astype(jnp.int32)
    ew = edge_weight.astype(f32)

    e = src.shape[0]
    per_worker = -(-e // (NW * CHUNK)) * CHUNK
    chunks = per_worker // CHUNK
    pad = NW * per_worker - e
    srcp = jnp.pad(src, (0, pad)).reshape(NW, chunks, CHUNK)
    dstp = jnp.pad(dst, (0, pad)).reshape(NW, chunks, CHUNK)
    ewp = jnp.pad(ew, (0, pad)).reshape(NW, chunks, CHUNK)

    degp = _make_deg(chunks)(dstp, ewp)
    degp_col = degp.reshape(NC, NPAD, 1)

    invwh = jnp.stack([1.0 / jnp.asarray(W, f32),
                       1.0 / jnp.asarray(H, f32)]).reshape(1, 2)
    scores_c = scores.astype(f32).reshape(N, 1)
    labels_c = labels.astype(jnp.int32).reshape(N, 1)
    b1 = mlp_b1.reshape(1, D)
    b2 = mlp_b2.reshape(1, D)
    gb0 = gcn_b0.reshape(1, D)
    gb1 = gcn_b1.reshape(1, D)
    gb2 = gcn_b2.reshape(1, D)
    hb = head_b.reshape(1, 4)

    edge = _make_edge(chunks)
    x0, y0 = _front(boxes, scores_c, labels_c, invwh, emb,
                    mlp_w1[:7], mlp_w1[7:], b1, mlp_w2, b2,
                    degp_col, gcn_w0)
    z0 = edge(y0, srcp, dstp, ewp)
    x1, y1 = _mid(z0, y0, x0, degp_col, gb0, gcn_w1)
    z1 = edge(y1, srcp, dstp, ewp)
    x2, y2 = _mid(z1, y1, x1, degp_col, gb1, gcn_w2)
    z2 = edge(y2, srcp, dstp, ewp)
    return _tail(z2, y2, x2, degp_col, gb2, head_w, hb)


# speedup vs baseline: 9.6790x; 9.6790x over previous
import functools

import jax
import jax.numpy as jnp
from jax import lax
from jax.experimental import pallas as pl
from jax.experimental.pallas import tpu as pltpu
from jax.experimental.pallas import tpu_sc as plsc

N = 10000
NPAD = 10240
D = 128
NCLS = 80
NC, NS = 2, 16
NW = NC * NS
CHUNK = 128
ROWS_PER_TILE = NPAD // NS

_mesh = plsc.VectorSubcoreMesh(core_axis_name="c", subcore_axis_name="s")


def _deg_body(chunks, dst_hbm, w_hbm, deg_hbm, dst_v, w_v, deg_sh, zbuf):
    c = lax.axis_index("c")
    s = lax.axis_index("s")
    wid = c * NS + s
    pltpu.sync_copy(dst_hbm.at[wid], dst_v)
    pltpu.sync_copy(w_hbm.at[wid], w_v)

    def zrow(i, carry):
        zbuf[pl.ds(i * 16, 16)] = jnp.zeros((16,), jnp.float32)
        return carry

    lax.fori_loop(0, ROWS_PER_TILE // 16, zrow, 0)
    pltpu.sync_copy(zbuf, deg_sh.at[pl.ds(s * ROWS_PER_TILE, ROWS_PER_TILE)])
    plsc.subcore_barrier()

    def chunk(j, carry):
        pltpu.sync_copy(w_v.at[j], deg_sh.at[dst_v.at[j]], add=True)
        return carry

    lax.fori_loop(0, chunks, chunk, 0)
    plsc.subcore_barrier()
    pltpu.sync_copy(deg_sh.at[pl.ds(s * ROWS_PER_TILE, ROWS_PER_TILE)],
                    deg_hbm.at[c, pl.ds(s * ROWS_PER_TILE, ROWS_PER_TILE)])


def _make_deg(chunks):
    return pl.kernel(
        functools.partial(_deg_body, chunks),
        out_type=jax.ShapeDtypeStruct((NC, NPAD), jnp.float32),
        mesh=_mesh,
        scratch_types=[
            pltpu.VMEM((chunks, CHUNK), jnp.int32),
            pltpu.VMEM((chunks, CHUNK), jnp.float32),
            pltpu.VMEM_SHARED((NPAD,), jnp.float32),
            pltpu.VMEM((ROWS_PER_TILE,), jnp.float32),
        ],
    )


def _edge_body(chunks, y_hbm, src_hbm, dst_hbm, w_hbm, z_hbm,
               src_v, dst_v, w_v, rbuf, acc, gsem):
    c = lax.axis_index("c")
    s = lax.axis_index("s")
    wid = c * NS + s
    pltpu.sync_copy(src_hbm.at[wid], src_v)
    pltpu.sync_copy(dst_hbm.at[wid], dst_v)
    pltpu.sync_copy(w_hbm.at[wid], w_v)

    def zrow(i, carry):
        for t in range(8):
            rbuf[i, pl.ds(t * 16, 16)] = jnp.zeros((16,), jnp.float32)
        return carry

    lax.fori_loop(0, CHUNK, zrow, 0)
    for k in range(ROWS_PER_TILE // CHUNK):
        pltpu.sync_copy(
            rbuf, acc.at[pl.ds(s * ROWS_PER_TILE + k * CHUNK, CHUNK)])
    plsc.subcore_barrier()

    def chunk(j, carry):
        pltpu.async_copy(y_hbm.at[src_v.at[j]], rbuf, gsem).wait()

        def scale(k16, c2):
            wv = w_v[j, pl.ds(k16 * 16, 16)]
            for i in range(16):
                wk = wv[i]
                r = k16 * 16 + i
                for t in range(8):
                    sl = pl.ds(t * 16, 16)
                    rbuf[r, sl] = rbuf[r, sl] * wk
            return c2

        lax.fori_loop(0, CHUNK // 16, scale, 0)
        pltpu.sync_copy(rbuf, acc.at[dst_v.at[j]], add=True)
        return carry

    lax.fori_loop(0, chunks, chunk, 0)
    plsc.subcore_barrier()
    pltpu.sync_copy(acc.at[pl.ds(s * ROWS_PER_TILE, ROWS_PER_TILE)],
                    z_hbm.at[c, pl.ds(s * ROWS_PER_TILE, ROWS_PER_TILE)])


def _make_edge(chunks):
    return pl.kernel(
        functools.partial(_edge_body, chunks),
        out_type=jax.ShapeDtypeStruct((NC, NPAD, D), jnp.float32),
        mesh=_mesh,
        scratch_types=[
            pltpu.VMEM((chunks, CHUNK), jnp.int32),
            pltpu.VMEM((chunks, CHUNK), jnp.int32),
            pltpu.VMEM((chunks, CHUNK), jnp.float32),
            pltpu.VMEM((CHUNK, D), jnp.float32),
            pltpu.VMEM_SHARED((NPAD, D), jnp.float32),
            pltpu.SemaphoreType.DMA,
        ],
    )


BLK = 1000
NBLK = N // BLK


def _row_spec(shape):
    return pl.BlockSpec(shape, lambda i: (i,) + (0,) * (len(shape) - 1))


def _full_spec(shape):
    return pl.BlockSpec(shape, lambda i: (0,) * len(shape))


def _prep_body(degp_r, dinv_out):
    deg = lax.dot_general(degp_r[...], jnp.ones((NC, D), jnp.float32),
                          (((0,), (0,)), ((), ())),
                          preferred_element_type=jnp.float32) + 1.0
    dinv_out[...] = jnp.where(deg > 0, lax.rsqrt(deg), 0.0)[:N]


_prep = pl.pallas_call(
    _prep_body,
    out_shape=jax.ShapeDtypeStruct((N, D), jnp.float32),
)


def _front_body(nodes_r, invwh_r, emb_r, w1g_r, w1e_r,
                b1_r, w2_r, b2_r, dinv_r, g0w_r, x_out, y0_out):
    nd = nodes_r[...]
    x1, y1, x2, y2 = nd[:, 0:1], nd[:, 1:2], nd[:, 2:3], nd[:, 3:4]
    w = jnp.maximum(x2 - x1, 1.0)
    h = jnp.maximum(y2 - y1, 1.0)
    inv = invwh_r[...]
    invW, invH = inv[0:1, 0:1], inv[0:1, 1:2]
    cxn = (x1 + x2) * 0.5 * invW
    cyn = (y1 + y2) * 0.5 * invH
    wn = w * invW
    hn = h * invH
    cols = (cxn, cyn, wn, hn, wn * hn, w / (h + 1e-6), nd[:, 4:5])
    g = w1g_r[...]
    pre = cols[0] * g[0:1]
    for t in range(1, 7):
        pre = pre + cols[t] * g[t:t + 1]
    lab = nd[:, 5:6].astype(jnp.int32)
    io = lax.broadcasted_iota(jnp.int32, (BLK, NCLS), 1)
    oh = jnp.where(io == lab, 1.0, 0.0)
    embw = jnp.dot(emb_r[...], w1e_r[...], preferred_element_type=jnp.float32)
    pre = pre + jnp.dot(oh, embw, preferred_element_type=jnp.float32) + b1_r[...]
    xx = jnp.maximum(pre, 0.0)
    xx = jnp.maximum(
        jnp.dot(xx, w2_r[...], preferred_element_type=jnp.float32) + b2_r[...],
        0.0)
    x_out[...] = xx
    y0_out[...] = jnp.dot(xx, g0w_r[...],
                          preferred_element_type=jnp.float32) * dinv_r[...]


_front = pl.pallas_call(
    _front_body,
    grid=(NBLK,),
    in_specs=[
        _row_spec((BLK, 8)),
        _full_spec((1, 2)),
        _full_spec((NCLS, 16)),
        _full_spec((7, D)),
        _full_spec((16, D)),
        _full_spec((1, D)),
        _full_spec((D, D)),
        _full_spec((1, D)),
        _row_spec((BLK, D)),
        _full_spec((D, D)),
    ],
    out_specs=(_row_spec((BLK, D)), _row_spec((BLK, D))),
    out_shape=(jax.ShapeDtypeStruct((N, D), jnp.float32),
               jax.ShapeDtypeStruct((N, D), jnp.float32)),
)


def _mid_body(zp_r, yp_r, x_r, dinv_r, b_r, wn_r, xn_out, yn_out):
    dinv = dinv_r[...]
    zp = zp_r[...]
    z = zp[0] + zp[1]
    conv = dinv * (z + yp_r[...]) + b_r[...]
    xn = jnp.maximum(conv, 0.0) + x_r[...]
    xn_out[...] = xn
    yn_out[...] = jnp.dot(xn, wn_r[...],
                          preferred_element_type=jnp.float32) * dinv


_mid = pl.pallas_call(
    _mid_body,
    grid=(NBLK,),
    in_specs=[
        pl.BlockSpec((NC, BLK, D), lambda i: (0, i, 0)),
        _row_spec((BLK, D)),
        _row_spec((BLK, D)),
        _row_spec((BLK, D)),
        _full_spec((1, D)),
        _full_spec((D, D)),
    ],
    out_specs=(_row_spec((BLK, D)), _row_spec((BLK, D))),
    out_shape=(jax.ShapeDtypeStruct((N, D), jnp.float32),
               jax.ShapeDtypeStruct((N, D), jnp.float32)),
)


def _tail_body(zp_r, yp_r, x_r, dinv_r, b_r, hw_r, hb_r, out):
    dinv = dinv_r[...]
    zp = zp_r[...]
    z = zp[0] + zp[1]
    conv = dinv * (z + yp_r[...]) + b_r[...]
    xn = jnp.maximum(conv, 0.0) + x_r[...]
    out[...] = jnp.dot(xn, hw_r[...],
                       preferred_element_type=jnp.float32) + hb_r[...]


_tail = pl.pallas_call(
    _tail_body,
    grid=(NBLK,),
    in_specs=[
        pl.BlockSpec((NC, BLK, D), lambda i: (0, i, 0)),
        _row_spec((BLK, D)),
        _row_spec((BLK, D)),
        _row_spec((BLK, D)),
        _full_spec((1, D)),
        _full_spec((D, 4)),
        _full_spec((1, 4)),
    ],
    out_specs=_row_spec((BLK, 4)),
    out_shape=jax.ShapeDtypeStruct((N, 4), jnp.float32),
)


def kernel(boxes, scores, labels, H, W, edge_index, edge_weight,
           emb, mlp_w1, mlp_b1, mlp_w2, mlp_b2,
           gcn_w0, gcn_b0, gcn_w1, gcn_b1, gcn_w2, gcn_b2,
           head_w, head_b):
    f32 = jnp.float32
    src = edge_index[0].astype(jnp.int32)
    dst = edge_index[1].astype(jnp.int32)
    ew = edge_weight.astype(f32)

    e = src.shape[0]
    per_worker = -(-e // (NW * CHUNK)) * CHUNK
    chunks = per_worker // CHUNK
    pad = NW * per_worker - e
    srcp = jnp.pad(src, (0, pad)).reshape(NW, chunks, CHUNK)
    dstp = jnp.pad(dst, (0, pad)).reshape(NW, chunks, CHUNK)
    ewp = jnp.pad(ew, (0, pad)).reshape(NW, chunks, CHUNK)

    degp = _make_deg(chunks)(dstp, ewp)

    invwh = jnp.stack([1.0 / jnp.asarray(W, f32),
                       1.0 / jnp.asarray(H, f32)]).reshape(1, 2)
    nodes = jnp.concatenate(
        [boxes.astype(f32), scores.astype(f32).reshape(N, 1),
         labels.astype(f32).reshape(N, 1), jnp.zeros((N, 2), f32)], axis=1)
    b1 = mlp_b1.reshape(1, D)
    b2 = mlp_b2.reshape(1, D)
    gb0 = gcn_b0.reshape(1, D)
    gb1 = gcn_b1.reshape(1, D)
    gb2 = gcn_b2.reshape(1, D)
    hb = head_b.reshape(1, 4)

    edge = _make_edge(chunks)
    dinvb = _prep(degp)
    x0, y0 = _front(nodes, invwh, emb,
                    mlp_w1[:7], mlp_w1[7:], b1, mlp_w2, b2,
                    dinvb, gcn_w0)
    z0 = edge(y0, srcp, dstp, ewp)
    x1, y1 = _mid(z0, y0, x0, dinvb, gb0, gcn_w1)
    z1 = edge(y1, srcp, dstp, ewp)
    x2, y2 = _mid(z1, y1, x1, dinvb, gb1, gcn_w2)
    z2 = edge(y2, srcp, dstp, ewp)
    return _tail(z2, y2, x2, dinvb, gb2, head_w, hb)
